# baseline (speedup 1.0000x reference)
import numpy as np

import jax
import jax.numpy as jnp
from jax.experimental import pallas as pl
from jax.experimental.pallas import tpu as pltpu
from jax.sharding import PartitionSpec as P

_TOPK = 2
_FB = 512


def _moe_kernel(off_ref, x_ref, g_ref, w1_ref, w2_ref, o_ref, cw_ref):
    e = pl.program_id(0)
    f = pl.program_id(1)

    @pl.when((e == 0) & (f == 0))
    def _router():
        x = x_ref[...]
        logits = jnp.dot(x, g_ref[...], preferred_element_type=jnp.float32)
        m = jnp.max(logits, axis=-1, keepdims=True)
        ex = jnp.exp(logits - m)
        p = ex / jnp.sum(ex, axis=-1, keepdims=True)
        lane = jax.lax.broadcasted_iota(jnp.int32, p.shape, 1)
        m1 = jnp.max(p, axis=-1, keepdims=True)
        i1 = jnp.min(jnp.where(p == m1, lane, p.shape[1]), axis=-1, keepdims=True)
        p2 = jnp.where(lane == i1, -1.0, p)
        m2 = jnp.max(p2, axis=-1, keepdims=True)
        i2 = jnp.min(jnp.where(p2 == m2, lane, p.shape[1]), axis=-1, keepdims=True)
        cw = jnp.where(lane == i1, m1, jnp.where(lane == i2, m2, 0.0))
        cw_ref[...] = cw / (m1 + m2)
        o_ref[...] = jnp.zeros_like(o_ref)

    cw = cw_ref[...]
    lane = jax.lax.broadcasted_iota(jnp.int32, cw.shape, 1)
    scale = jnp.sum(jnp.where(lane == e + off_ref[0], cw, 0.0),
                    axis=-1, keepdims=True)

    x = x_ref[...].astype(jnp.bfloat16)
    h = jnp.dot(x, w1_ref[0].astype(jnp.bfloat16),
                preferred_element_type=jnp.float32)
    h = h * jax.nn.sigmoid(h)
    h = h * scale
    o_ref[...] += jnp.dot(h.astype(jnp.bfloat16),
                          w2_ref[0].astype(jnp.bfloat16),
                          preferred_element_type=jnp.float32)


def _moe_shard(off, hidden_states, gate_w, w1, w2):
    T, D = hidden_states.shape
    E = gate_w.shape[1]
    e_local, _, F = w1.shape
    nf = F // _FB

    return pl.pallas_call(
        _moe_kernel,
        grid=(e_local, nf),
        in_specs=[
            pl.BlockSpec(memory_space=pltpu.SMEM),
            pl.BlockSpec((T, D), lambda e, f: (0, 0)),
            pl.BlockSpec((D, E), lambda e, f: (0, 0)),
            pl.BlockSpec((1, D, _FB), lambda e, f: (e, 0, f)),
            pl.BlockSpec((1, _FB, D), lambda e, f: (e, f, 0)),
        ],
        out_specs=pl.BlockSpec((T, D), lambda e, f: (0, 0)),
        out_shape=jax.ShapeDtypeStruct((T, D), jnp.float32),
        scratch_shapes=[pltpu.VMEM((T, E), jnp.float32)],
        compiler_params=pltpu.CompilerParams(
            dimension_semantics=("arbitrary", "arbitrary"),
        ),
    )(off, hidden_states, gate_w, w1, w2)


@jax.jit
def kernel(hidden_states, gate_w, w1, w2):
    E = gate_w.shape[1]
    devs = jax.devices()
    n = 2 if len(devs) >= 2 and E % 2 == 0 else 1
    mesh = jax.sharding.Mesh(np.array(devs[:n]), ("x",))

    def shard_fn(x, g, w1s, w2s):
        off = (jax.lax.axis_index("x") * (E // n)).astype(jnp.int32)
        partial = _moe_shard(off.reshape(1), x, g, w1s, w2s)
        return jax.lax.psum(partial, "x")

    return jax.shard_map(
        shard_fn,
        mesh=mesh,
        in_specs=(P(None, None), P(None, None),
                  P("x", None, None), P("x", None, None)),
        out_specs=P(None, None),
        check_vma=False,
    )(hidden_states, gate_w, w1, w2)


# speedup vs baseline: 3.7593x; 3.7593x over previous
import functools

import jax
import jax.numpy as jnp
from jax.experimental import pallas as pl
from jax.experimental.pallas import tpu as pltpu
from jax.experimental.pallas import tpu_sc as plsc

_TOPK = 2
_FB = 512
_LANES = 16


def _logits_kernel(x_ref, g_ref, o_ref):
    lg = jnp.dot(x_ref[...], g_ref[...], preferred_element_type=jnp.float32)
    o_ref[...] = lg.T


def _router_sc(logits_t):
    E, T = logits_t.shape
    n_chunks = T // _LANES
    mesh = plsc.VectorSubcoreMesh(core_axis_name="c", subcore_axis_name="s")

    @functools.partial(
        pl.kernel,
        out_type=jax.ShapeDtypeStruct((E, T), jnp.float32),
        mesh=mesh,
        scratch_types=[
            pltpu.VMEM((E, _LANES), jnp.float32),
            pltpu.VMEM((E, _LANES), jnp.float32),
        ],
    )
    def k(lg_hbm, cw_hbm, lg_v, out_v):
        wid = jax.lax.axis_index("s") * 2 + jax.lax.axis_index("c")

        @pl.when(wid < n_chunks)
        def _():
            base = wid * _LANES
            for r in range(E):
                pltpu.sync_copy(lg_hbm.at[r, pl.ds(base, _LANES)], lg_v.at[r])
            m1 = lg_v[0]
            i1 = jnp.zeros((_LANES,), jnp.int32)
            m2 = jnp.full((_LANES,), -jnp.inf, jnp.float32)
            i2 = jnp.zeros((_LANES,), jnp.int32)
            for r in range(1, E):
                v = lg_v[r]
                rr = jnp.full((_LANES,), r, jnp.int32)
                new1 = v > m1
                new2 = v > m2
                m2 = jnp.where(new1, m1, jnp.where(new2, v, m2))
                i2 = jnp.where(new1, i1, jnp.where(new2, rr, i2))
                m1 = jnp.where(new1, v, m1)
                i1 = jnp.where(new1, rr, i1)
            w_hi = 1.0 / (1.0 + jnp.exp(m2 - m1))
            w_lo = 1.0 - w_hi
            zero = jnp.zeros((_LANES,), jnp.float32)
            for r in range(E):
                rr = jnp.full((_LANES,), r, jnp.int32)
                out_v[r] = (jnp.where(i1 == rr, w_hi, zero)
                            + jnp.where(i2 == rr, w_lo, zero))
            for r in range(E):
                pltpu.sync_copy(out_v.at[r], cw_hbm.at[r, pl.ds(base, _LANES)])

    return k(logits_t)


def _moe_kernel(x_ref, cwt_ref, w1_ref, w2_ref, o_ref, cw_ref):
    e = pl.program_id(0)
    f = pl.program_id(1)

    @pl.when((e == 0) & (f == 0))
    def _init():
        cw_ref[...] = cwt_ref[...].T
        o_ref[...] = jnp.zeros_like(o_ref)

    cw = cw_ref[...]
    lane = jax.lax.broadcasted_iota(jnp.int32, cw.shape, 1)
    scale = jnp.sum(jnp.where(lane == e, cw, 0.0), axis=-1, keepdims=True)

    x = x_ref[...].astype(jnp.bfloat16)
    h = jnp.dot(x, w1_ref[0].astype(jnp.bfloat16),
                preferred_element_type=jnp.float32)
    h = h * jax.nn.sigmoid(h)
    h = h * scale
    o_ref[...] += jnp.dot(h.astype(jnp.bfloat16),
                          w2_ref[0].astype(jnp.bfloat16),
                          preferred_element_type=jnp.float32)


@jax.jit
def kernel(hidden_states, gate_w, w1, w2):
    T, D = hidden_states.shape
    E = gate_w.shape[1]
    F = w1.shape[2]
    nf = F // _FB

    logits_t = pl.pallas_call(
        _logits_kernel,
        out_shape=jax.ShapeDtypeStruct((E, T), jnp.float32),
    )(hidden_states, gate_w)

    cwt = _router_sc(logits_t)

    return pl.pallas_call(
        _moe_kernel,
        grid=(E, nf),
        in_specs=[
            pl.BlockSpec((T, D), lambda e, f: (0, 0)),
            pl.BlockSpec((E, T), lambda e, f: (0, 0)),
            pl.BlockSpec((1, D, _FB), lambda e, f: (e, 0, f)),
            pl.BlockSpec((1, _FB, D), lambda e, f: (e, f, 0)),
        ],
        out_specs=pl.BlockSpec((T, D), lambda e, f: (0, 0)),
        out_shape=jax.ShapeDtypeStruct((T, D), jnp.float32),
        scratch_shapes=[pltpu.VMEM((T, E), jnp.float32)],
        compiler_params=pltpu.CompilerParams(
            dimension_semantics=("arbitrary", "arbitrary"),
        ),
    )(hidden_states, cwt, w1, w2)


# speedup vs baseline: 4.1762x; 1.1109x over previous
import jax
import jax.numpy as jnp
from jax.experimental import pallas as pl
from jax.experimental.pallas import tpu as pltpu

_TOPK = 2
_FB = 512
_NBUF = 4


def _moe_kernel(x_ref, g_ref, w1_hbm, w2_hbm, o_ref,
                cw_ref, w1b, w2b, sem1, sem2):
    s = pl.program_id(0)
    n_steps = pl.num_programs(0)
    nf = w1_hbm.shape[2] // _FB

    def w1_copy(t, slot):
        e_t = t // nf
        f_t = jax.lax.rem(t, nf)
        return pltpu.make_async_copy(
            w1_hbm.at[e_t, :, pl.ds(f_t * _FB, _FB)], w1b.at[slot],
            sem1.at[slot])

    def w2_copy(t, slot):
        e_t = t // nf
        f_t = jax.lax.rem(t, nf)
        return pltpu.make_async_copy(
            w2_hbm.at[e_t, pl.ds(f_t * _FB, _FB), :], w2b.at[slot],
            sem2.at[slot])

    @pl.when(s == 0)
    def _prologue():
        for j in range(_NBUF):
            w1_copy(j, j).start()
            w2_copy(j, j).start()
        x = x_ref[...]
        logits = jnp.dot(x, g_ref[...], preferred_element_type=jnp.float32)
        m = jnp.max(logits, axis=-1, keepdims=True)
        ex = jnp.exp(logits - m)
        p = ex / jnp.sum(ex, axis=-1, keepdims=True)
        lane = jax.lax.broadcasted_iota(jnp.int32, p.shape, 1)
        m1 = jnp.max(p, axis=-1, keepdims=True)
        i1 = jnp.min(jnp.where(p == m1, lane, p.shape[1]), axis=-1, keepdims=True)
        p2 = jnp.where(lane == i1, -1.0, p)
        m2 = jnp.max(p2, axis=-1, keepdims=True)
        i2 = jnp.min(jnp.where(p2 == m2, lane, p.shape[1]), axis=-1, keepdims=True)
        cw = jnp.where(lane == i1, m1, jnp.where(lane == i2, m2, 0.0))
        cw_ref[...] = cw / (m1 + m2)
        o_ref[...] = jnp.zeros_like(o_ref)

    e = s // nf
    slot = jax.lax.rem(s, _NBUF)
    w1_copy(s, slot).wait()
    w2_copy(s, slot).wait()

    cw = cw_ref[...]
    lane = jax.lax.broadcasted_iota(jnp.int32, cw.shape, 1)
    scale = jnp.sum(jnp.where(lane == e, cw, 0.0), axis=-1, keepdims=True)

    x = x_ref[...].astype(jnp.bfloat16)
    h = jnp.dot(x, w1b[slot].astype(jnp.bfloat16),
                preferred_element_type=jnp.float32)
    h = h * jax.nn.sigmoid(h)
    h = h * scale
    o_ref[...] += jnp.dot(h.astype(jnp.bfloat16),
                          w2b[slot].astype(jnp.bfloat16),
                          preferred_element_type=jnp.float32)

    @pl.when(s + _NBUF < n_steps)
    def _refill():
        w1_copy(s + _NBUF, slot).start()
        w2_copy(s + _NBUF, slot).start()


@jax.jit
def kernel(hidden_states, gate_w, w1, w2):
    T, D = hidden_states.shape
    E = gate_w.shape[1]
    F = w1.shape[2]
    nf = F // _FB

    return pl.pallas_call(
        _moe_kernel,
        grid=(E * nf,),
        in_specs=[
            pl.BlockSpec((T, D), lambda s: (0, 0)),
            pl.BlockSpec((D, E), lambda s: (0, 0)),
            pl.BlockSpec(memory_space=pl.ANY),
            pl.BlockSpec(memory_space=pl.ANY),
        ],
        out_specs=pl.BlockSpec((T, D), lambda s: (0, 0)),
        out_shape=jax.ShapeDtypeStruct((T, D), jnp.float32),
        scratch_shapes=[
            pltpu.VMEM((T, E), jnp.float32),
            pltpu.VMEM((_NBUF, D, _FB), jnp.float32),
            pltpu.VMEM((_NBUF, _FB, D), jnp.float32),
            pltpu.SemaphoreType.DMA((_NBUF,)),
            pltpu.SemaphoreType.DMA((_NBUF,)),
        ],
        compiler_params=pltpu.CompilerParams(
            dimension_semantics=("arbitrary",),
        ),
    )(hidden_states, gate_w, w1, w2)


# speedup vs baseline: 4.1965x; 1.0049x over previous
import jax
import jax.numpy as jnp
from jax.experimental import pallas as pl
from jax.experimental.pallas import tpu as pltpu

_TOPK = 2
_FB = 512
_NBUF = 4


def _moe_kernel(x_ref, g_ref, w1_hbm, w2_hbm, o_ref,
                cw_ref, w1b, w2b, sem1, sem2):
    s = pl.program_id(0)
    n_steps = pl.num_programs(0)
    nf = w1_hbm.shape[2] // _FB

    def w1_copy(t, slot):
        e_t = t // nf
        f_t = jax.lax.rem(t, nf)
        return pltpu.make_async_copy(
            w1_hbm.at[e_t, :, pl.ds(f_t * _FB, _FB)], w1b.at[slot],
            sem1.at[slot])

    def w2_copy(t, slot):
        e_t = t // nf
        f_t = jax.lax.rem(t, nf)
        return pltpu.make_async_copy(
            w2_hbm.at[e_t, pl.ds(f_t * _FB, _FB), :], w2b.at[slot],
            sem2.at[slot])

    @pl.when(s == 0)
    def _prologue():
        for j in range(_NBUF - 1):
            w1_copy(j, j).start()
            w2_copy(j, j).start()
        x = x_ref[...]
        logits = jnp.dot(x, g_ref[...], preferred_element_type=jnp.float32)
        m = jnp.max(logits, axis=-1, keepdims=True)
        ex = jnp.exp(logits - m)
        p = ex / jnp.sum(ex, axis=-1, keepdims=True)
        lane = jax.lax.broadcasted_iota(jnp.int32, p.shape, 1)
        m1 = jnp.max(p, axis=-1, keepdims=True)
        i1 = jnp.min(jnp.where(p == m1, lane, p.shape[1]), axis=-1, keepdims=True)
        p2 = jnp.where(lane == i1, -1.0, p)
        m2 = jnp.max(p2, axis=-1, keepdims=True)
        i2 = jnp.min(jnp.where(p2 == m2, lane, p.shape[1]), axis=-1, keepdims=True)
        cw = jnp.where(lane == i1, m1, jnp.where(lane == i2, m2, 0.0))
        cw_ref[...] = cw / (m1 + m2)
        o_ref[...] = jnp.zeros_like(o_ref)

    @pl.when(s + _NBUF - 1 < n_steps)
    def _refill():
        t = s + _NBUF - 1
        w1_copy(t, jax.lax.rem(t, _NBUF)).start()
        w2_copy(t, jax.lax.rem(t, _NBUF)).start()

    e = s // nf
    slot = jax.lax.rem(s, _NBUF)
    w1_copy(s, slot).wait()
    w2_copy(s, slot).wait()

    cw = cw_ref[...]
    lane = jax.lax.broadcasted_iota(jnp.int32, cw.shape, 1)
    scale = jnp.sum(jnp.where(lane == e, cw, 0.0), axis=-1, keepdims=True)

    x = x_ref[...].astype(jnp.bfloat16)
    h = jnp.dot(x, w1b[slot].astype(jnp.bfloat16),
                preferred_element_type=jnp.float32)
    h = h * jax.nn.sigmoid(h)
    h = h * scale
    o_ref[...] += jnp.dot(h.astype(jnp.bfloat16),
                          w2b[slot].astype(jnp.bfloat16),
                          preferred_element_type=jnp.float32)


@jax.jit
def kernel(hidden_states, gate_w, w1, w2):
    T, D = hidden_states.shape
    E = gate_w.shape[1]
    F = w1.shape[2]
    nf = F // _FB

    return pl.pallas_call(
        _moe_kernel,
        grid=(E * nf,),
        in_specs=[
            pl.BlockSpec((T, D), lambda s: (0, 0)),
            pl.BlockSpec((D, E), lambda s: (0, 0)),
            pl.BlockSpec(memory_space=pl.ANY),
            pl.BlockSpec(memory_space=pl.ANY),
        ],
        out_specs=pl.BlockSpec((T, D), lambda s: (0, 0)),
        out_shape=jax.ShapeDtypeStruct((T, D), jnp.float32),
        scratch_shapes=[
            pltpu.VMEM((T, E), jnp.float32),
            pltpu.VMEM((_NBUF, D, _FB), jnp.float32),
            pltpu.VMEM((_NBUF, _FB, D), jnp.float32),
            pltpu.SemaphoreType.DMA((_NBUF,)),
            pltpu.SemaphoreType.DMA((_NBUF,)),
        ],
        compiler_params=pltpu.CompilerParams(
            dimension_semantics=("arbitrary",),
        ),
    )(hidden_states, gate_w, w1, w2)


# speedup vs baseline: 4.2602x; 1.0152x over previous
import jax
import jax.numpy as jnp
from jax.experimental import pallas as pl
from jax.experimental.pallas import tpu as pltpu

_TOPK = 2


def _moe_kernel(x_ref, g_ref, w1_ref, w2_ref, o_ref, cw_ref):
    e = pl.program_id(0)
    f = pl.program_id(1)

    @pl.when((e == 0) & (f == 0))
    def _router():
        x = x_ref[...]
        logits = jnp.dot(x, g_ref[...], preferred_element_type=jnp.float32)
        m = jnp.max(logits, axis=-1, keepdims=True)
        ex = jnp.exp(logits - m)
        p = ex / jnp.sum(ex, axis=-1, keepdims=True)
        lane = jax.lax.broadcasted_iota(jnp.int32, p.shape, 1)
        m1 = jnp.max(p, axis=-1, keepdims=True)
        i1 = jnp.min(jnp.where(p == m1, lane, p.shape[1]), axis=-1, keepdims=True)
        p2 = jnp.where(lane == i1, -1.0, p)
        m2 = jnp.max(p2, axis=-1, keepdims=True)
        i2 = jnp.min(jnp.where(p2 == m2, lane, p.shape[1]), axis=-1, keepdims=True)
        cw = jnp.where(lane == i1, m1, jnp.where(lane == i2, m2, 0.0))
        cw_ref[...] = cw / (m1 + m2)
        o_ref[...] = jnp.zeros_like(o_ref)

    cw = cw_ref[...]
    lane = jax.lax.broadcasted_iota(jnp.int32, cw.shape, 1)
    scale = jnp.sum(jnp.where(lane == e, cw, 0.0), axis=-1, keepdims=True)

    x = x_ref[...].astype(jnp.bfloat16)
    h = jnp.dot(x, w1_ref[0].astype(jnp.bfloat16),
                preferred_element_type=jnp.float32)
    h = h * jax.nn.sigmoid(h)
    h = h * scale
    o_ref[...] += jnp.dot(h.astype(jnp.bfloat16),
                          w2_ref[0].astype(jnp.bfloat16),
                          preferred_element_type=jnp.float32)


@jax.jit
def kernel(hidden_states, gate_w, w1, w2):
    T, D = hidden_states.shape
    E = gate_w.shape[1]
    F = w1.shape[2]
    FB = 512
    nf = F // FB

    grid = (E, nf)
    return pl.pallas_call(
        _moe_kernel,
        grid=grid,
        in_specs=[
            pl.BlockSpec((T, D), lambda e, f: (0, 0)),
            pl.BlockSpec((D, E), lambda e, f: (0, 0)),
            pl.BlockSpec((1, D, FB), lambda e, f: (e, 0, f)),
            pl.BlockSpec((1, FB, D), lambda e, f: (e, f, 0)),
        ],
        out_specs=pl.BlockSpec((T, D), lambda e, f: (0, 0)),
        out_shape=jax.ShapeDtypeStruct((T, D), jnp.float32),
        scratch_shapes=[pltpu.VMEM((T, E), jnp.float32)],
        compiler_params=pltpu.CompilerParams(
            dimension_semantics=("arbitrary", "arbitrary"),
        ),
    )(hidden_states, gate_w, w1, w2)
